# baseline (speedup 1.0000x reference)
import jax
import jax.numpy as jnp
from jax import lax
from jax.experimental import pallas as pl
from jax.experimental.pallas import tpu as pltpu
from jax.experimental.pallas import tpu_sc as plsc

_NC = 2
_NS = 16
_NW = _NC * _NS
_L = 16
_CAP = 131072
_BATCH = 16384
_LR = 0.05

_WIN = _CAP // _NW
_WSHIFT = 12
_CPT = _BATCH // _NW
_NJ = _CPT // 128
_UNROLL = 16


def _body(cur_hbm, cand_hbm, latest_hbm, gap_hbm,
          prob_hbm, out_latest_hbm, out_gap_hbm,
          cand_v, lat_w, gap_w, cg_v, latp, gapp, prob_v, curv,
          sem_a, sem_g, sem_c):
    c = lax.axis_index("c")
    s = lax.axis_index("s")
    w = c * _NS + s
    wbase = w * _WIN

    stc = pltpu.async_copy(cand_hbm, cand_v, sem_c)
    st1 = pltpu.async_copy(latest_hbm.at[pl.ds(wbase, _WIN)], lat_w, sem_a)
    st2 = pltpu.async_copy(gap_hbm.at[pl.ds(wbase, _WIN)], gap_w, sem_a)
    pltpu.sync_copy(cur_hbm, curv)
    stc.wait()

    cbase = w * _CPT
    gathers = []
    for j in range(_NJ):
        gathers.append(pltpu.async_copy(
            latest_hbm.at[cand_v.at[pl.ds(cbase + j * 128, 128)]],
            latp.at[j], sem_g))
        gathers.append(pltpu.async_copy(
            gap_hbm.at[cand_v.at[pl.ds(cbase + j * 128, 128)]],
            gapp.at[j], sem_g))
    for g in gathers:
        g.wait()

    cur = curv[...]
    zero = jnp.zeros((_L,), jnp.int32)

    for i in range(_CPT // _L):
        j, k = divmod(i, 128 // _L)
        sl = pl.ds(k * _L, _L)
        lat = latp[j, sl]
        gp = gapp[j, sl]
        delta = (cur - lat).astype(jnp.float32)
        coef = jnp.where(lat == 0, 1.0, _LR).astype(jnp.float32)
        cg = (1.0 - _LR) * gp + coef * delta
        prob_v[pl.ds(i * _L, _L)] = 1.0 / cg
    pr = pltpu.async_copy(prob_v, prob_hbm.at[pl.ds(cbase, _CPT)], sem_g)
    st1.wait()
    st2.wait()

    @plsc.parallel_loop(0, _BATCH // _L, unroll=_UNROLL)
    def pass1(it):
        off = it * _L
        idx = cand_v[pl.ds(off, _L)]
        mask = (idx >> _WSHIFT) == w
        lidx = jnp.where(mask, idx - wbase, zero)
        lat = plsc.load_gather(lat_w, [lidx], mask=mask)
        gp = plsc.load_gather(gap_w, [lidx], mask=mask)
        delta = (cur - lat).astype(jnp.float32)
        coef = jnp.where(lat == 0, 1.0, _LR).astype(jnp.float32)
        cg_v[pl.ds(off, _L)] = (1.0 - _LR) * gp + coef * delta

    @plsc.parallel_loop(0, _BATCH // _L, unroll=_UNROLL)
    def pass2(it):
        off = it * _L
        idx = cand_v[pl.ds(off, _L)]
        mask = (idx >> _WSHIFT) == w
        lidx = jnp.where(mask, idx - wbase, zero)
        plsc.store_scatter(lat_w, [lidx], cur, mask=mask)
        plsc.store_scatter(gap_w, [lidx], cg_v[pl.ds(off, _L)],
                           mask=mask)

    fl1 = pltpu.async_copy(lat_w, out_latest_hbm.at[pl.ds(wbase, _WIN)],
                           sem_a)
    fl2 = pltpu.async_copy(gap_w, out_gap_hbm.at[pl.ds(wbase, _WIN)], sem_a)
    fl1.wait()
    fl2.wait()
    pr.wait()


@jax.jit
def _sc_call(cur_vec, candidate_ids, latest_step, step_gap):
    mesh = plsc.VectorSubcoreMesh(core_axis_name="c", subcore_axis_name="s")
    f = pl.kernel(
        _body,
        out_type=(
            jax.ShapeDtypeStruct((_BATCH,), jnp.float32),
            jax.ShapeDtypeStruct((_CAP,), jnp.int32),
            jax.ShapeDtypeStruct((_CAP,), jnp.float32),
        ),
        mesh=mesh,
        compiler_params=pltpu.CompilerParams(needs_layout_passes=False),
        scratch_types=(
            pltpu.VMEM((_BATCH,), jnp.int32),
            pltpu.VMEM((_WIN,), jnp.int32),
            pltpu.VMEM((_WIN,), jnp.float32),
            pltpu.VMEM((_BATCH,), jnp.float32),
            pltpu.VMEM((_NJ, 128), jnp.int32),
            pltpu.VMEM((_NJ, 128), jnp.float32),
            pltpu.VMEM((_CPT,), jnp.float32),
            pltpu.VMEM((_L,), jnp.int32),
            pltpu.SemaphoreType.DMA,
            pltpu.SemaphoreType.DMA,
            pltpu.SemaphoreType.DMA,
        ),
    )
    return f(cur_vec, candidate_ids, latest_step, step_gap)


def kernel(cur_step, candidate_ids, latest_step, step_gap):
    cur_vec = jnp.full((_L,), cur_step, dtype=jnp.int32)
    return _sc_call(cur_vec, candidate_ids, latest_step, step_gap)


# speedup vs baseline: 1.2379x; 1.2379x over previous
import jax
import jax.numpy as jnp
from jax import lax
from jax.experimental import pallas as pl
from jax.experimental.pallas import tpu as pltpu
from jax.experimental.pallas import tpu_sc as plsc

_NC = 2
_NS = 16
_NW = _NC * _NS
_L = 16
_CAP = 131072
_BATCH = 16384
_LR = 0.05

_WIN = _CAP // _NW
_WSHIFT = 12
_CPT = _BATCH // _NW
_NJ = _CPT // 128
_UNROLL = 8


def _body(cur_hbm, cand_hbm, latest_hbm, gap_hbm,
          prob_hbm, out_latest_hbm, out_gap_hbm,
          cand_v, lat_w, gap_w, cg_v, latp, gapp, prob_v, curv,
          sem_a, sem_g, sem_c):
    c = lax.axis_index("c")
    s = lax.axis_index("s")
    w = c * _NS + s
    wbase = w * _WIN

    stc = pltpu.async_copy(cand_hbm, cand_v, sem_c)
    st1 = pltpu.async_copy(latest_hbm.at[pl.ds(wbase, _WIN)], lat_w, sem_a)
    st2 = pltpu.async_copy(gap_hbm.at[pl.ds(wbase, _WIN)], gap_w, sem_a)
    pltpu.sync_copy(cur_hbm, curv)
    stc.wait()

    cbase = w * _CPT
    gathers = []
    for j in range(_NJ):
        gathers.append(pltpu.async_copy(
            latest_hbm.at[cand_v.at[pl.ds(cbase + j * 128, 128)]],
            latp.at[j], sem_g))
        gathers.append(pltpu.async_copy(
            gap_hbm.at[cand_v.at[pl.ds(cbase + j * 128, 128)]],
            gapp.at[j], sem_g))
    for g in gathers:
        g.wait()

    cur = curv[...]
    zero = jnp.zeros((_L,), jnp.int32)

    for i in range(_CPT // _L):
        j, k = divmod(i, 128 // _L)
        sl = pl.ds(k * _L, _L)
        lat = latp[j, sl]
        gp = gapp[j, sl]
        delta = (cur - lat).astype(jnp.float32)
        coef = jnp.where(lat == 0, 1.0, _LR).astype(jnp.float32)
        cg = (1.0 - _LR) * gp + coef * delta
        prob_v[pl.ds(i * _L, _L)] = 1.0 / cg
    pr = pltpu.async_copy(prob_v, prob_hbm.at[pl.ds(cbase, _CPT)], sem_g)
    st1.wait()
    st2.wait()

    @plsc.parallel_loop(0, _BATCH // _L, unroll=_UNROLL)
    def pass1(it):
        off = it * _L
        idx = cand_v[pl.ds(off, _L)]
        mask = (idx >> _WSHIFT) == w
        lidx = jnp.where(mask, idx - wbase, zero)
        lat = plsc.load_gather(lat_w, [lidx], mask=mask)
        gp = plsc.load_gather(gap_w, [lidx], mask=mask)
        delta = (cur - lat).astype(jnp.float32)
        coef = jnp.where(lat == 0, 1.0, _LR).astype(jnp.float32)
        cg_v[pl.ds(off, _L)] = (1.0 - _LR) * gp + coef * delta

    @plsc.parallel_loop(0, _BATCH // _L, unroll=_UNROLL)
    def pass2(it):
        off = it * _L
        idx = cand_v[pl.ds(off, _L)]
        mask = (idx >> _WSHIFT) == w
        lidx = jnp.where(mask, idx - wbase, zero)
        plsc.store_scatter(lat_w, [lidx], cur, mask=mask)
        plsc.store_scatter(gap_w, [lidx], cg_v[pl.ds(off, _L)],
                           mask=mask)

    fl1 = pltpu.async_copy(lat_w, out_latest_hbm.at[pl.ds(wbase, _WIN)],
                           sem_a)
    fl2 = pltpu.async_copy(gap_w, out_gap_hbm.at[pl.ds(wbase, _WIN)], sem_a)
    fl1.wait()
    fl2.wait()
    pr.wait()


@jax.jit
def _sc_call(cur_vec, candidate_ids, latest_step, step_gap):
    mesh = plsc.VectorSubcoreMesh(core_axis_name="c", subcore_axis_name="s")
    f = pl.kernel(
        _body,
        out_type=(
            jax.ShapeDtypeStruct((_BATCH,), jnp.float32),
            jax.ShapeDtypeStruct((_CAP,), jnp.int32),
            jax.ShapeDtypeStruct((_CAP,), jnp.float32),
        ),
        mesh=mesh,
        compiler_params=pltpu.CompilerParams(needs_layout_passes=False),
        scratch_types=(
            pltpu.VMEM((_BATCH,), jnp.int32),
            pltpu.VMEM((_WIN,), jnp.int32),
            pltpu.VMEM((_WIN,), jnp.float32),
            pltpu.VMEM((_BATCH,), jnp.float32),
            pltpu.VMEM((_NJ, 128), jnp.int32),
            pltpu.VMEM((_NJ, 128), jnp.float32),
            pltpu.VMEM((_CPT,), jnp.float32),
            pltpu.VMEM((_L,), jnp.int32),
            pltpu.SemaphoreType.DMA,
            pltpu.SemaphoreType.DMA,
            pltpu.SemaphoreType.DMA,
        ),
    )
    return f(cur_vec, candidate_ids, latest_step, step_gap)


def kernel(cur_step, candidate_ids, latest_step, step_gap):
    cur_vec = jnp.full((_L,), cur_step, dtype=jnp.int32)
    return _sc_call(cur_vec, candidate_ids, latest_step, step_gap)


# speedup vs baseline: 1.3244x; 1.0699x over previous
import jax
import jax.numpy as jnp
from jax import lax
from jax.experimental import pallas as pl
from jax.experimental.pallas import tpu as pltpu
from jax.experimental.pallas import tpu_sc as plsc

_NC = 2
_NS = 16
_NW = _NC * _NS
_L = 16
_CAP = 131072
_BATCH = 16384
_LR = 0.05

_WIN = _CAP // _NW
_WSHIFT = 12
_CPT = _BATCH // _NW
_NJ = _CPT // 128
_UNROLL = 8


def _body(cur_hbm, cand_hbm, latest_hbm, gap_hbm,
          prob_hbm, out_latest_hbm, out_gap_hbm,
          cand_v, lat_w, gap_w, cg_v, lidx_v, latp, gapp, prob_v, curv,
          sem_a, sem_g, sem_c):
    c = lax.axis_index("c")
    s = lax.axis_index("s")
    w = c * _NS + s
    wbase = w * _WIN

    stc = pltpu.async_copy(cand_hbm, cand_v, sem_c)
    st1 = pltpu.async_copy(latest_hbm.at[pl.ds(wbase, _WIN)],
                           lat_w.at[pl.ds(0, _WIN)], sem_a)
    st2 = pltpu.async_copy(gap_hbm.at[pl.ds(wbase, _WIN)],
                           gap_w.at[pl.ds(0, _WIN)], sem_a)
    pltpu.sync_copy(cur_hbm, curv)
    stc.wait()

    cbase = w * _CPT
    gathers = []
    for j in range(_NJ):
        gathers.append(pltpu.async_copy(
            latest_hbm.at[cand_v.at[pl.ds(cbase + j * 128, 128)]],
            latp.at[j], sem_g))
        gathers.append(pltpu.async_copy(
            gap_hbm.at[cand_v.at[pl.ds(cbase + j * 128, 128)]],
            gapp.at[j], sem_g))

    cur = curv[...]
    trashv = _WIN + lax.broadcasted_iota(jnp.int32, (_L,), 0)
    st1.wait()
    st2.wait()

    @plsc.parallel_loop(0, _BATCH // _L, unroll=_UNROLL)
    def pass1(it):
        off = it * _L
        idx = cand_v[pl.ds(off, _L)]
        mask = (idx >> _WSHIFT) == w
        lidx = jnp.where(mask, idx - wbase, trashv)
        lat = plsc.load_gather(lat_w, [lidx])
        gp = plsc.load_gather(gap_w, [lidx])
        delta = (cur - lat).astype(jnp.float32)
        coef = jnp.where(lat == 0, 1.0, _LR).astype(jnp.float32)
        cg_v[pl.ds(off, _L)] = (1.0 - _LR) * gp + coef * delta
        lidx_v[pl.ds(off, _L)] = lidx

    for g in gathers:
        g.wait()
    for i in range(_CPT // _L):
        j, k = divmod(i, 128 // _L)
        sl = pl.ds(k * _L, _L)
        lat = latp[j, sl]
        gp = gapp[j, sl]
        delta = (cur - lat).astype(jnp.float32)
        coef = jnp.where(lat == 0, 1.0, _LR).astype(jnp.float32)
        cg = (1.0 - _LR) * gp + coef * delta
        prob_v[pl.ds(i * _L, _L)] = 1.0 / cg
    pr = pltpu.async_copy(prob_v, prob_hbm.at[pl.ds(cbase, _CPT)], sem_g)

    @plsc.parallel_loop(0, _BATCH // _L, unroll=_UNROLL)
    def pass2(it):
        off = it * _L
        lidx = lidx_v[pl.ds(off, _L)]
        plsc.store_scatter(lat_w, [lidx], cur)
        plsc.store_scatter(gap_w, [lidx], cg_v[pl.ds(off, _L)])

    fl1 = pltpu.async_copy(lat_w.at[pl.ds(0, _WIN)],
                           out_latest_hbm.at[pl.ds(wbase, _WIN)], sem_a)
    fl2 = pltpu.async_copy(gap_w.at[pl.ds(0, _WIN)],
                           out_gap_hbm.at[pl.ds(wbase, _WIN)], sem_a)
    fl1.wait()
    fl2.wait()
    pr.wait()


@jax.jit
def _sc_call(cur_vec, candidate_ids, latest_step, step_gap):
    mesh = plsc.VectorSubcoreMesh(core_axis_name="c", subcore_axis_name="s")
    f = pl.kernel(
        _body,
        out_type=(
            jax.ShapeDtypeStruct((_BATCH,), jnp.float32),
            jax.ShapeDtypeStruct((_CAP,), jnp.int32),
            jax.ShapeDtypeStruct((_CAP,), jnp.float32),
        ),
        mesh=mesh,
        compiler_params=pltpu.CompilerParams(needs_layout_passes=False),
        scratch_types=(
            pltpu.VMEM((_BATCH,), jnp.int32),
            pltpu.VMEM((_WIN + _L,), jnp.int32),
            pltpu.VMEM((_WIN + _L,), jnp.float32),
            pltpu.VMEM((_BATCH,), jnp.float32),
            pltpu.VMEM((_BATCH,), jnp.int32),
            pltpu.VMEM((_NJ, 128), jnp.int32),
            pltpu.VMEM((_NJ, 128), jnp.float32),
            pltpu.VMEM((_CPT,), jnp.float32),
            pltpu.VMEM((_L,), jnp.int32),
            pltpu.SemaphoreType.DMA,
            pltpu.SemaphoreType.DMA,
            pltpu.SemaphoreType.DMA,
        ),
    )
    return f(cur_vec, candidate_ids, latest_step, step_gap)


def kernel(cur_step, candidate_ids, latest_step, step_gap):
    cur_vec = jnp.full((_L,), cur_step, dtype=jnp.int32)
    return _sc_call(cur_vec, candidate_ids, latest_step, step_gap)
